# baseline (speedup 1.0000x reference)
import jax
import jax.numpy as jnp
from jax.experimental import pallas as pl
from jax.experimental.pallas import tpu as pltpu

B, N, D, H = 256, 512, 256, 256
BB = 16


def _graph_summary_kernel(x_ref, w1_ref, b1_ref, w2gt_ref,
                          w2n_ref, b2n_ref, out_ref):
    xb = x_ref[...]

    z = jnp.dot(xb, w1_ref[...], preferred_element_type=jnp.float32)
    z = z + b1_ref[...]
    a = z * (1.0 + jnp.tanh(z))
    a1 = a[:, :H]
    a2 = a[:, H:]
    gate = jnp.sum(a1 * w2gt_ref[...], axis=1, keepdims=True)

    rows = []
    for b in range(BB):
        g = gate[b * N:(b + 1) * N, :]
        m = jnp.max(g)
        e = jnp.exp2(g - m)

        denom = jnp.sum(e)
        pooled = jax.lax.dot_general(
            e, a2[b * N:(b + 1) * N, :],
            (((0,), (0,)), ((), ())),
            preferred_element_type=jnp.float32)
        rows.append(pooled / (denom + 1e-16))
    pooled_all = jnp.concatenate(rows, axis=0)
    out_ref[...] = jnp.dot(pooled_all, w2n_ref[...],
                           preferred_element_type=jnp.float32) + b2n_ref[...]


@jax.jit
def kernel(x, W1g, b1g, W2g, b2g, W1n, b1n, W2n, b2n):
    flat = x.reshape(B * N, D)
    W1 = jnp.concatenate([0.5 * W1g, 0.5 * W1n], axis=1)
    b1 = jnp.concatenate([0.5 * b1g, 0.5 * b1n]).reshape(1, 2 * H)
    grid = (B // BB,)
    full = lambda i: (0, 0)
    out = pl.pallas_call(
        _graph_summary_kernel,
        grid=grid,
        in_specs=[
            pl.BlockSpec((BB * N, D), lambda i: (i, 0)),
            pl.BlockSpec((D, 2 * H), full),
            pl.BlockSpec((1, 2 * H), full),
            pl.BlockSpec((1, H), full),
            pl.BlockSpec((H, H), full),
            pl.BlockSpec((1, H), full),
        ],
        out_specs=pl.BlockSpec((BB, H), lambda i: (i, 0)),
        out_shape=jax.ShapeDtypeStruct((B, H), jnp.float32),
        compiler_params=pltpu.CompilerParams(
            dimension_semantics=("parallel",),
        ),
    )(flat, W1, b1, (1.4426950408889634 * W2g).reshape(1, H),
      W2n, b2n.reshape(1, H))
    return out
